# baseline (speedup 1.0000x reference)
import functools

import jax
import jax.numpy as jnp
from jax import lax
from jax.experimental import pallas as pl
from jax.experimental.pallas import tpu as pltpu
from jax.experimental.pallas import tpu_sc as plsc

_B = 16384
_F = 64
_L = 16
_NC = 2
_NS = 16
_NW = _NC * _NS
_BW = _B // _NW
_CHUNK = 128
_NCHUNK = _BW // _CHUNK
_PASS = 256
_NPASS = _BW // _PASS
_CPP = _PASS // _CHUNK
_KV = _F // _L
_CB = 4096

_V_U = 1000000
_V_I = 100000
_S_U = (_V_U // 2) // _CB * _CB
_S_I = (_V_I // 2) // _CB * _CB
_OUT_U = _V_U - _S_U
_OUT_I = _V_I - _S_I


_BLK = 512
_P = 64
_PR = _P // 2
_NB_U = (_V_U + _BLK - 1) // _BLK


def _mxu_gather_body(su_ref, tab_ref, o_ref):
    vals = su_ref[0, 0, :]
    a = tab_ref[...]
    col0 = pl.program_id(0) * _BLK
    rows = lax.broadcasted_iota(jnp.int32, (_BLK, _P), 0) + col0
    valsb = lax.broadcast_in_dim(vals, (_BLK, _P), (1,))
    oneh = (valsb == rows).astype(jnp.float32)
    g = lax.dot_general(oneh, a, (((0,), (1,)), ((), ())),
                        preferred_element_type=jnp.float32)
    o_ref[...] = jnp.concatenate([g[:_PR], g[_PR:]], axis=1)


def _mxu_bucket_gather(tabT, su_pad):
    return pl.pallas_call(
        _mxu_gather_body,
        grid=(_NB_U,),
        in_specs=[
            pl.BlockSpec((1, 1, _P), lambda j: (j, 0, 0)),
            pl.BlockSpec((_F, _BLK), lambda j: (0, j)),
        ],
        out_specs=pl.BlockSpec((_PR, 2 * _F), lambda j: (j, 0)),
        out_shape=jax.ShapeDtypeStruct((_NB_U * _PR, 2 * _F), jnp.float32),
    )(su_pad, tabT)


def _bucketize(u):
    iota = jnp.arange(_B, dtype=jnp.int32)
    order = jnp.argsort(u)
    su = u[order]
    blk = su >> 9
    prev = jnp.concatenate([jnp.full((1,), -1, jnp.int32), blk[:-1]])
    seg_start = lax.associative_scan(
        jnp.maximum, jnp.where(blk != prev, iota, 0))
    rank = jnp.minimum(iota - seg_start, _P - 1)
    pos = blk * _P + rank
    su_pad = jnp.full((_NB_U * _P,), -1, jnp.int32).at[pos].set(su)
    grow_s = blk * _PR + (rank & (_PR - 1))
    off_s = (rank >> 5) << 6
    grow = jnp.zeros((_B,), jnp.int32).at[order].set(grow_s)
    off = jnp.zeros((_B,), jnp.int32).at[order].set(off_s)
    return su_pad.reshape(_NB_U, 1, _P), grow, off


def _pair_transpose_body(a_ref, b_ref, o_ref):
    a = a_ref[...]
    b = b_ref[...]
    o_ref[:, 0:_F] = a.T
    o_ref[:, _F:2 * _F] = b.T


def _pair_table(tabT, split, out_rows):
    nblk_s = split // _CB
    grid = (out_rows + _CB - 1) // _CB
    return pl.pallas_call(
        _pair_transpose_body,
        grid=(grid,),
        in_specs=[
            pl.BlockSpec((_F, _CB), lambda j: (0, j)),
            pl.BlockSpec((_F, _CB), lambda j, s=nblk_s: (0, j + s)),
        ],
        out_specs=pl.BlockSpec((_CB, 2 * _F), lambda j: (j, 0)),
        out_shape=jax.ShapeDtypeStruct((out_rows, 2 * _F), jnp.float32),
    )(tabT, tabT)


def _gmf_body(uidx_hbm, iidx_hbm, pk_hbm, utab_hbm, itab_hbm, w_hbm, b_hbm,
              out_hbm, uidx_v, iidx_v, urows_v, irows_v, w_v, b_v, out_v,
              pk_v, sem):
    wid = lax.axis_index("s") * _NC + lax.axis_index("c")
    row0 = wid * _NCHUNK
    base = wid * _BW

    pltpu.sync_copy(uidx_hbm.at[pl.ds(row0, _NCHUNK)], uidx_v)
    pltpu.sync_copy(iidx_hbm.at[pl.ds(row0, _NCHUNK)], iidx_v)
    pltpu.sync_copy(pk_hbm.at[pl.ds(base, _BW)], pk_v)
    pltpu.sync_copy(w_hbm, w_v)
    pltpu.sync_copy(b_hbm, b_v)

    wv = [w_v[pl.ds(k * _L, _L)] for k in range(_KV)]
    bias = b_v[...]
    lane = lax.broadcasted_iota(jnp.int32, (_L,), 0)
    perms = [lane ^ (_L >> (p + 1)) for p in range(4)]

    for p in range(_NPASS):
        copies = []
        for jj in range(_CPP):
            j = p * _CPP + jj
            copies.append(pltpu.async_copy(
                utab_hbm.at[uidx_v.at[j]],
                urows_v.at[pl.ds(jj * _CHUNK, _CHUNK)], sem))
            copies.append(pltpu.async_copy(
                itab_hbm.at[iidx_v.at[j]],
                irows_v.at[pl.ds(jj * _CHUNK, _CHUNK)], sem))
        for c in copies:
            c.wait()

        def group(g, carry):
            res = bias
            pkvec = pk_v[pl.ds(p * _PASS + g * _L, _L)]
            for r in range(_L):
                b = g * _L + r
                pk = pkvec[r]
                uo = pk & 0xFF
                io = (pk >> 8) & 0xFF
                acc = (urows_v[b, pl.ds(uo, _L)]
                       * irows_v[b, pl.ds(io, _L)]) * wv[0]
                for k in range(1, _KV):
                    acc = acc + (urows_v[b, pl.ds(uo + k * _L, _L)]
                                 * irows_v[b, pl.ds(io + k * _L, _L)]) * wv[k]
                for q in perms:
                    acc = acc + acc.at[q].get(mode="promise_in_bounds")
                res = jnp.where(lane == r, res + acc, res)
            out_v[pl.ds(p * _PASS + g * _L, _L)] = res
            return carry

        lax.fori_loop(0, _PASS // _L, group, 0)

    pltpu.sync_copy(out_v, out_hbm.at[pl.ds(base, _BW)])


@jax.jit
def _gmf(user_indices, item_indices, user_table, item_table, linear_w, linear_b):
    u = user_indices.astype(jnp.int32)
    i = item_indices.astype(jnp.int32)
    su_pad, grow, offu = _bucketize(u)
    uidx = grow.reshape(_B // _CHUNK, _CHUNK)
    iidx = jnp.where(i < _S_I, i, i - _S_I).reshape(_B // _CHUNK, _CHUNK)
    pk = offu | (jnp.where(i < _S_I, 0, _F) << 8)
    utab = _mxu_bucket_gather(user_table.T, su_pad)
    itab = _pair_table(item_table.T, _S_I, _OUT_I)
    w = linear_w.reshape(_F)
    bias = jnp.full((_L,), linear_b[0], dtype=jnp.float32)
    mesh = plsc.VectorSubcoreMesh(core_axis_name="c", subcore_axis_name="s",
                                  num_cores=_NC, num_subcores=_NS)
    out = pl.kernel(
        _gmf_body,
        out_type=jax.ShapeDtypeStruct((_B,), jnp.float32),
        mesh=mesh,
        scratch_types=[
            pltpu.VMEM((_NCHUNK, _CHUNK), jnp.int32),
            pltpu.VMEM((_NCHUNK, _CHUNK), jnp.int32),
            pltpu.VMEM((_PASS, 2 * _F), jnp.float32),
            pltpu.VMEM((_PASS, 2 * _F), jnp.float32),
            pltpu.VMEM((_F,), jnp.float32),
            pltpu.VMEM((_L,), jnp.float32),
            pltpu.VMEM((_BW,), jnp.float32),
            pltpu.VMEM((_BW,), jnp.int32),
            pltpu.SemaphoreType.DMA,
        ],
    )(uidx, iidx, pk, utab, itab, w, bias)
    return out.reshape(_B, 1)


def kernel(user_indices, item_indices, user_table, item_table, linear_w, linear_b):
    return _gmf(user_indices, item_indices, user_table, item_table,
                linear_w, linear_b)


# speedup vs baseline: 5.2275x; 5.2275x over previous
import functools

import jax
import jax.numpy as jnp
from jax import lax
from jax.experimental import pallas as pl
from jax.experimental.pallas import tpu as pltpu
from jax.experimental.pallas import tpu_sc as plsc

_B = 16384
_F = 64
_L = 16
_NC = 2
_NS = 16
_NW = _NC * _NS
_BW = _B // _NW
_CHUNK = 128
_NCHUNK = _BW // _CHUNK
_PASS = 256
_NPASS = _BW // _PASS
_CPP = _PASS // _CHUNK
_KV = _F // _L
_CB = 8192

_V_U = 1000000
_V_I = 100000
_S_U = (_V_U // 2) // _CB * _CB
_S_I = (_V_I // 2) // _CB * _CB
_OUT_U = _V_U - _S_U
_OUT_I = _V_I - _S_I


def _pair_transpose_body(a_ref, b_ref, o_ref):
    a = a_ref[...]
    b = b_ref[...]
    o_ref[:, 0:_F] = a.T
    o_ref[:, _F:2 * _F] = b.T


def _pair_table(tabT, split, out_rows):
    nblk_s = split // _CB
    grid = (out_rows + _CB - 1) // _CB
    return pl.pallas_call(
        _pair_transpose_body,
        grid=(grid,),
        in_specs=[
            pl.BlockSpec((_F, _CB), lambda j: (0, j)),
            pl.BlockSpec((_F, _CB), lambda j, s=nblk_s: (0, j + s)),
        ],
        out_specs=pl.BlockSpec((_CB, 2 * _F), lambda j: (j, 0)),
        out_shape=jax.ShapeDtypeStruct((out_rows, 2 * _F), jnp.float32),
    )(tabT, tabT)


def _gmf_body(uidx_hbm, iidx_hbm, pk_hbm, utab_hbm, itab_hbm, w_hbm, b_hbm,
              out_hbm, uidx_v, iidx_v, urows_v, irows_v, w_v, b_v, out_v,
              pk_v, sem):
    wid = lax.axis_index("s") * _NC + lax.axis_index("c")
    row0 = wid * _NCHUNK
    base = wid * _BW

    pltpu.sync_copy(uidx_hbm.at[pl.ds(row0, _NCHUNK)], uidx_v)
    pltpu.sync_copy(iidx_hbm.at[pl.ds(row0, _NCHUNK)], iidx_v)
    pltpu.sync_copy(pk_hbm.at[pl.ds(base, _BW)], pk_v)
    pltpu.sync_copy(w_hbm, w_v)
    pltpu.sync_copy(b_hbm, b_v)

    wv = [w_v[pl.ds(k * _L, _L)] for k in range(_KV)]
    bias = b_v[...]
    lane = lax.broadcasted_iota(jnp.int32, (_L,), 0)
    perms = [lane ^ (_L >> (p + 1)) for p in range(4)]

    for p in range(_NPASS):
        copies = []
        for jj in range(_CPP):
            j = p * _CPP + jj
            copies.append(pltpu.async_copy(
                utab_hbm.at[uidx_v.at[j]],
                urows_v.at[pl.ds(jj * _CHUNK, _CHUNK)], sem))
            copies.append(pltpu.async_copy(
                itab_hbm.at[iidx_v.at[j]],
                irows_v.at[pl.ds(jj * _CHUNK, _CHUNK)], sem))
        for c in copies:
            c.wait()

        def group(g, carry):
            res = bias
            pkvec = pk_v[pl.ds(p * _PASS + g * _L, _L)]
            for r in range(_L):
                b = g * _L + r
                pk = pkvec[r]
                uo = pk & 0xFF
                io = (pk >> 8) & 0xFF
                acc = (urows_v[b, pl.ds(uo, _L)]
                       * irows_v[b, pl.ds(io, _L)]) * wv[0]
                for k in range(1, _KV):
                    acc = acc + (urows_v[b, pl.ds(uo + k * _L, _L)]
                                 * irows_v[b, pl.ds(io + k * _L, _L)]) * wv[k]
                for q in perms:
                    acc = acc + acc.at[q].get(mode="promise_in_bounds")
                res = jnp.where(lane == r, res + acc, res)
            out_v[pl.ds(p * _PASS + g * _L, _L)] = res
            return carry

        lax.fori_loop(0, _PASS // _L, group, 0)

    pltpu.sync_copy(out_v, out_hbm.at[pl.ds(base, _BW)])


@jax.jit
def _gmf(user_indices, item_indices, user_table, item_table, linear_w, linear_b):
    u = user_indices.astype(jnp.int32)
    i = item_indices.astype(jnp.int32)
    uidx = jnp.where(u < _S_U, u, u - _S_U).reshape(_B // _CHUNK, _CHUNK)
    iidx = jnp.where(i < _S_I, i, i - _S_I).reshape(_B // _CHUNK, _CHUNK)
    pk = (jnp.where(u < _S_U, 0, _F)
          | (jnp.where(i < _S_I, 0, _F) << 8))
    utab = _pair_table(user_table.T, _S_U, _OUT_U)
    itab = _pair_table(item_table.T, _S_I, _OUT_I)
    w = linear_w.reshape(_F)
    bias = jnp.full((_L,), linear_b[0], dtype=jnp.float32)
    mesh = plsc.VectorSubcoreMesh(core_axis_name="c", subcore_axis_name="s",
                                  num_cores=_NC, num_subcores=_NS)
    out = pl.kernel(
        _gmf_body,
        out_type=jax.ShapeDtypeStruct((_B,), jnp.float32),
        mesh=mesh,
        scratch_types=[
            pltpu.VMEM((_NCHUNK, _CHUNK), jnp.int32),
            pltpu.VMEM((_NCHUNK, _CHUNK), jnp.int32),
            pltpu.VMEM((_PASS, 2 * _F), jnp.float32),
            pltpu.VMEM((_PASS, 2 * _F), jnp.float32),
            pltpu.VMEM((_F,), jnp.float32),
            pltpu.VMEM((_L,), jnp.float32),
            pltpu.VMEM((_BW,), jnp.float32),
            pltpu.VMEM((_BW,), jnp.int32),
            pltpu.SemaphoreType.DMA,
        ],
    )(uidx, iidx, pk, utab, itab, w, bias)
    return out.reshape(_B, 1)


def kernel(user_indices, item_indices, user_table, item_table, linear_w, linear_b):
    return _gmf(user_indices, item_indices, user_table, item_table,
                linear_w, linear_b)


# speedup vs baseline: 5.3487x; 1.0232x over previous
import functools

import jax
import jax.numpy as jnp
from jax import lax
from jax.experimental import pallas as pl
from jax.experimental.pallas import tpu as pltpu
from jax.experimental.pallas import tpu_sc as plsc

_B = 16384
_F = 64
_L = 16
_NC = 2
_NS = 16
_NW = _NC * _NS
_BW = _B // _NW
_CHUNK = 128
_NCHUNK = _BW // _CHUNK
_PASS = 256
_NPASS = _BW // _PASS
_CPP = _PASS // _CHUNK
_KV = _F // _L
_CB = 16384

_V_U = 1000000
_V_I = 100000
_S_U = (_V_U // 2) // _CB * _CB
_S_I = (_V_I // 2) // _CB * _CB
_OUT_U = _V_U - _S_U
_OUT_I = _V_I - _S_I


def _pair_transpose_body(a_ref, b_ref, o_ref):
    a = a_ref[...]
    b = b_ref[...]
    o_ref[:, 0:_F] = a.T
    o_ref[:, _F:2 * _F] = b.T


def _pair_table(tabT, split, out_rows):
    nblk_s = split // _CB
    grid = (out_rows + _CB - 1) // _CB
    return pl.pallas_call(
        _pair_transpose_body,
        grid=(grid,),
        in_specs=[
            pl.BlockSpec((_F, _CB), lambda j: (0, j)),
            pl.BlockSpec((_F, _CB), lambda j, s=nblk_s: (0, j + s)),
        ],
        out_specs=pl.BlockSpec((_CB, 2 * _F), lambda j: (j, 0)),
        out_shape=jax.ShapeDtypeStruct((out_rows, 2 * _F), jnp.float32),
    )(tabT, tabT)


def _gmf_body(uidx_hbm, iidx_hbm, pk_hbm, utab_hbm, itab_hbm, w_hbm, b_hbm,
              out_hbm, uidx_v, iidx_v, urows_v, irows_v, w_v, b_v, out_v,
              pk_v, sem):
    wid = lax.axis_index("s") * _NC + lax.axis_index("c")
    row0 = wid * _NCHUNK
    base = wid * _BW

    pltpu.sync_copy(uidx_hbm.at[pl.ds(row0, _NCHUNK)], uidx_v)
    pltpu.sync_copy(iidx_hbm.at[pl.ds(row0, _NCHUNK)], iidx_v)
    pltpu.sync_copy(pk_hbm.at[pl.ds(base, _BW)], pk_v)
    pltpu.sync_copy(w_hbm, w_v)
    pltpu.sync_copy(b_hbm, b_v)

    wv = [w_v[pl.ds(k * _L, _L)] for k in range(_KV)]
    bias = b_v[...]
    lane = lax.broadcasted_iota(jnp.int32, (_L,), 0)
    perms = [lane ^ (_L >> (p + 1)) for p in range(4)]

    for p in range(_NPASS):
        copies = []
        for jj in range(_CPP):
            j = p * _CPP + jj
            copies.append(pltpu.async_copy(
                utab_hbm.at[uidx_v.at[j]],
                urows_v.at[pl.ds(jj * _CHUNK, _CHUNK)], sem))
            copies.append(pltpu.async_copy(
                itab_hbm.at[iidx_v.at[j]],
                irows_v.at[pl.ds(jj * _CHUNK, _CHUNK)], sem))
        for c in copies:
            c.wait()

        def group(g, carry):
            res = bias
            pkvec = pk_v[pl.ds(p * _PASS + g * _L, _L)]
            for r in range(_L):
                b = g * _L + r
                pk = pkvec[r]
                uo = pk & 0xFF
                io = (pk >> 8) & 0xFF
                acc = (urows_v[b, pl.ds(uo, _L)]
                       * irows_v[b, pl.ds(io, _L)]) * wv[0]
                for k in range(1, _KV):
                    acc = acc + (urows_v[b, pl.ds(uo + k * _L, _L)]
                                 * irows_v[b, pl.ds(io + k * _L, _L)]) * wv[k]
                for q in perms:
                    acc = acc + acc.at[q].get(mode="promise_in_bounds")
                res = jnp.where(lane == r, res + acc, res)
            out_v[pl.ds(p * _PASS + g * _L, _L)] = res
            return carry

        lax.fori_loop(0, _PASS // _L, group, 0)

    pltpu.sync_copy(out_v, out_hbm.at[pl.ds(base, _BW)])


@jax.jit
def _gmf(user_indices, item_indices, user_table, item_table, linear_w, linear_b):
    u = user_indices.astype(jnp.int32)
    i = item_indices.astype(jnp.int32)
    uidx = jnp.where(u < _S_U, u, u - _S_U).reshape(_B // _CHUNK, _CHUNK)
    iidx = jnp.where(i < _S_I, i, i - _S_I).reshape(_B // _CHUNK, _CHUNK)
    pk = (jnp.where(u < _S_U, 0, _F)
          | (jnp.where(i < _S_I, 0, _F) << 8))
    utab = _pair_table(user_table.T, _S_U, _OUT_U)
    itab = _pair_table(item_table.T, _S_I, _OUT_I)
    w = linear_w.reshape(_F)
    bias = jnp.full((_L,), linear_b[0], dtype=jnp.float32)
    mesh = plsc.VectorSubcoreMesh(core_axis_name="c", subcore_axis_name="s",
                                  num_cores=_NC, num_subcores=_NS)
    out = pl.kernel(
        _gmf_body,
        out_type=jax.ShapeDtypeStruct((_B,), jnp.float32),
        mesh=mesh,
        scratch_types=[
            pltpu.VMEM((_NCHUNK, _CHUNK), jnp.int32),
            pltpu.VMEM((_NCHUNK, _CHUNK), jnp.int32),
            pltpu.VMEM((_PASS, 2 * _F), jnp.float32),
            pltpu.VMEM((_PASS, 2 * _F), jnp.float32),
            pltpu.VMEM((_F,), jnp.float32),
            pltpu.VMEM((_L,), jnp.float32),
            pltpu.VMEM((_BW,), jnp.float32),
            pltpu.VMEM((_BW,), jnp.int32),
            pltpu.SemaphoreType.DMA,
        ],
    )(uidx, iidx, pk, utab, itab, w, bias)
    return out.reshape(_B, 1)


def kernel(user_indices, item_indices, user_table, item_table, linear_w, linear_b):
    return _gmf(user_indices, item_indices, user_table, item_table,
                linear_w, linear_b)
